# baseline (speedup 1.0000x reference)
import functools

import jax
import jax.numpy as jnp
from jax import lax
from jax.experimental import pallas as pl
from jax.experimental.pallas import tpu as pltpu
from jax.experimental.pallas import tpu_sc as plsc

_LANES = 16
_CH = 128
_NBUF = 3


@functools.lru_cache(maxsize=None)
def _make_gather(total_rows, feat, rows_per_batch, table_rows_per_batch):
    info = plsc.get_sparse_core_info()
    nc, ns = info.num_cores, info.num_subcores
    nw = nc * ns
    per_w = total_rows // nw
    n_chunks = per_w // _CH
    n_main = (n_chunks // _NBUF) * _NBUF
    assert n_chunks - n_main == 2
    assert rows_per_batch % per_w == 0
    mesh = plsc.VectorSubcoreMesh(core_axis_name="c", subcore_axis_name="s")

    @functools.partial(
        pl.kernel,
        mesh=mesh,
        out_type=jax.ShapeDtypeStruct((total_rows, feat), jnp.float32),
        scratch_types=[
            pltpu.VMEM((per_w,), jnp.int32),
            pltpu.VMEM((_NBUF, _CH, feat), jnp.float32),
            pltpu.SemaphoreType.DMA,
            pltpu.SemaphoreType.DMA,
            pltpu.SemaphoreType.DMA,
            pltpu.SemaphoreType.DMA,
            pltpu.SemaphoreType.DMA,
            pltpu.SemaphoreType.DMA,
        ],
    )
    def gather_kernel(idx_hbm, table_hbm, out_hbm, idx_v, rbuf,
                      sg0, sg1, sg2, ss0, ss1, ss2):
        wid = lax.axis_index("s") * nc + lax.axis_index("c")
        base = wid * per_w
        off = (base // rows_per_batch) * table_rows_per_batch

        pltpu.sync_copy(idx_hbm.at[pl.ds(base, per_w)], idx_v)

        sg = (sg0, sg1, sg2)
        ss = (ss0, ss1, ss2)

        def adjust(i):
            for j in range(_CH // _LANES):
                sl = pl.ds(i * _CH + j * _LANES, _LANES)
                idx_v[sl] = idx_v[sl] + off

        def gather_desc(i, bf):
            return pltpu.make_async_copy(
                table_hbm.at[idx_v.at[pl.ds(i * _CH, _CH)]],
                rbuf.at[bf], sg[bf])

        def store_desc(i, bf):
            return pltpu.make_async_copy(
                rbuf.at[bf], out_hbm.at[pl.ds(base + i * _CH, _CH)], ss[bf])

        adjust(0)
        gather_desc(0, 0).start()
        adjust(1)
        gather_desc(1, 1).start()

        def step(i, bf):
            gather_desc(i, bf).wait()
            nb = (bf + 2) % _NBUF

            @pl.when(i >= 1)
            def _():
                store_desc(i - 1, nb).wait()

            @pl.when(i + 2 < n_chunks)
            def _():
                adjust(i + 2)
                gather_desc(i + 2, nb).start()

            store_desc(i, bf).start()

        def loop_body(g, carry):
            for bf in range(_NBUF):
                step(_NBUF * g + bf, bf)
            return carry

        lax.fori_loop(0, n_main // _NBUF, loop_body, 0)
        step(n_main, n_main % _NBUF)
        step(n_main + 1, (n_main + 1) % _NBUF)
        store_desc(n_chunks - 1, (n_chunks - 1) % _NBUF).wait()

    return gather_kernel


def kernel(lcas, features):
    batch, length, feat = features.shape
    table = jnp.concatenate(
        [jnp.zeros((batch, 1, feat), features.dtype), features], axis=1
    ).reshape(batch * (length + 1), feat)
    idx = lcas.astype(jnp.int32).reshape(-1)
    total = batch * length * length
    out = _make_gather(total, feat, length * length, length + 1)(idx, table)
    return out.reshape(batch, length, length, feat)
